# baseline (speedup 1.0000x reference)
import jax
import jax.numpy as jnp
from jax import lax
from jax.experimental import pallas as pl
from jax.experimental.pallas import tpu as pltpu

jax.config.update("jax_enable_x64", True)

L_SEQ = 32768
ROWS = 256
COLS = 128
BYTE_BASE = 10


def _tc_body(marker_ref, tok_ref, out_ref):
    x = tok_ref[...]
    m = marker_ref[0, 0]
    row = lax.broadcasted_iota(jnp.int32, (ROWS, COLS), 0)
    col = lax.broadcasted_iota(jnp.int32, (ROWS, COLS), 1)
    idx = row * COLS + col
    scores = jnp.where(x == m, idx, jnp.int32(-1))
    pos = jnp.max(scores)
    found = pos >= 0
    pos0 = jnp.maximum(pos, 0)
    r0 = jnp.minimum((pos0 + 1) // COLS, ROWS - 2)
    win = tok_ref[pl.ds(r0, 2), :]
    wrow = lax.broadcasted_iota(jnp.int32, (2, COLS), 0)
    wcol = lax.broadcasted_iota(jnp.int32, (2, COLS), 1)
    widx = (r0 + wrow) * COLS + wcol
    value = jnp.int32(0)
    mults = (1, 256, 65536, 16777216)
    for k in range(4):
        t = jnp.clip(pos0 + jnp.int32(1 + k), 0, L_SEQ - 1)
        tok = jnp.max(jnp.where(widx == t, win, jnp.int32(0)))
        byte = jnp.clip(tok - jnp.int32(BYTE_BASE), 0, 255)
        value = value + byte * jnp.int32(mults[k])
    value = value * jnp.where(found, jnp.int32(1), jnp.int32(0))
    out_ref[0, 0] = value


def kernel(context_tokens, marker):
    tok32 = context_tokens[0].astype(jnp.int32).reshape(ROWS, COLS)
    marker_arr = jnp.asarray(marker, jnp.int32).reshape(1, 1)
    out = pl.pallas_call(
        _tc_body,
        out_shape=jax.ShapeDtypeStruct((1, 1), jnp.int32),
        in_specs=[
            pl.BlockSpec(memory_space=pltpu.SMEM),
            pl.BlockSpec(memory_space=pltpu.VMEM),
        ],
        out_specs=pl.BlockSpec(memory_space=pltpu.SMEM),
    )(marker_arr, tok32)
    return out[0, 0].astype(jnp.int64) & jnp.int64(4294967295)


# speedup vs baseline: 1.0012x; 1.0012x over previous
import jax
import jax.numpy as jnp
from jax import lax
from jax.experimental import pallas as pl
from jax.experimental.pallas import tpu as pltpu

jax.config.update("jax_enable_x64", True)

L_SEQ = 32768
ROWS = 256
COLS = 128
BYTE_BASE = 10


def _tc_body(marker_ref, tok_ref, out_ref):
    x = tok_ref[...]
    m = marker_ref[0, 0]
    row = lax.broadcasted_iota(jnp.int32, (ROWS, COLS), 0)
    col = lax.broadcasted_iota(jnp.int32, (ROWS, COLS), 1)
    idx = row * COLS + col
    scores = jnp.where(x == m, idx, jnp.int32(-1))
    pos = jnp.max(scores)
    found = pos >= 0
    pos0 = jnp.maximum(pos, 0)
    r0 = jnp.minimum((pos0 + 1) // COLS, ROWS - 2)
    win = tok_ref[pl.ds(r0, 2), :]
    wrow = lax.broadcasted_iota(jnp.int32, (2, COLS), 0)
    wcol = lax.broadcasted_iota(jnp.int32, (2, COLS), 1)
    widx = (r0 + wrow) * COLS + wcol
    value = jnp.int32(0)
    mults = (1, 256, 65536, 16777216)
    for k in range(4):
        t = jnp.clip(pos0 + jnp.int32(1 + k), 0, L_SEQ - 1)
        tok = jnp.max(jnp.where(widx == t, win, jnp.int32(0)))
        byte = jnp.clip(tok - jnp.int32(BYTE_BASE), 0, 255)
        value = value + byte * jnp.int32(mults[k])
    value = value * jnp.where(found, jnp.int32(1), jnp.int32(0))
    out_ref[0, 0] = value


def kernel(context_tokens, marker):
    tok32 = context_tokens[0].astype(jnp.int32).reshape(ROWS, COLS)
    marker_arr = jnp.asarray(marker, jnp.int32).reshape(1, 1)
    out = pl.pallas_call(
        _tc_body,
        out_shape=jax.ShapeDtypeStruct((1, 1), jnp.int32),
        in_specs=[
            pl.BlockSpec(memory_space=pltpu.SMEM),
            pl.BlockSpec(memory_space=pltpu.VMEM),
        ],
        out_specs=pl.BlockSpec(memory_space=pltpu.SMEM),
        compiler_params=pltpu.CompilerParams(
            allow_input_fusion=[False, True]),
    )(marker_arr, tok32)
    return out[0, 0].astype(jnp.int64) & jnp.int64(4294967295)
